# baseline (speedup 1.0000x reference)
import functools

import jax
import jax.numpy as jnp
from jax import lax
from jax.experimental import pallas as pl
from jax.experimental.pallas import tpu as pltpu
from jax.experimental.pallas import tpu_sc as plsc

NC = 2
NS = 16
NW = NC * NS

V = 1000000
D = 64
T = 50
S = 16384
NB = 5
GU = (T * (S // 128)) // NW


def _iota16():
    return lax.iota(jnp.int32, 16)


@jax.jit
def _gather_t(xs3, tl):
    mesh = plsc.VectorSubcoreMesh(core_axis_name="c", subcore_axis_name="s")

    @functools.partial(
        pl.kernel,
        mesh=mesh,
        out_type=jax.ShapeDtypeStruct((T, 8, S // 128, 8, 128), jnp.float32),
        scratch_types=[
            pltpu.VMEM((GU, 128), jnp.int32),
            pltpu.VMEM((NB, 128, 64), jnp.float32),
            pltpu.VMEM((NB, 8, 8, 129), jnp.float32),
            pltpu.SemaphoreType.DMA((NB,)),
            pltpu.SemaphoreType.DMA((NB,)),
        ],
        compiler_params=pltpu.CompilerParams(
            use_tc_tiling_on_sc=False, needs_layout_passes=False
        ),
    )
    def body(xs3_hbm, tl_hbm, out_hbm, idx_v, rows_v, rowsT_v, gsems, osems):
        wid = lax.axis_index("s") * NC + lax.axis_index("c")

        def unit(r):
            u = r * NW + wid
            return u // 128, u % 128

        def start_gather(b, r):
            pltpu.async_copy(
                tl_hbm.at[idx_v.at[r]], rows_v.at[b], gsems.at[b]
            )

        def wait_gather(b):
            pltpu.make_async_copy(
                tl_hbm.at[pl.ds(0, 128)], rows_v.at[b], gsems.at[b]
            ).wait()

        def start_out(b, r):
            t, w = unit(r)
            pltpu.async_copy(
                rowsT_v.at[b, :, :, pl.ds(0, 128)],
                out_hbm.at[t, :, w],
                osems.at[b],
            )

        def wait_out(b):
            pltpu.make_async_copy(
                rowsT_v.at[b, :, :, pl.ds(0, 128)],
                out_hbm.at[0, :, 0],
                osems.at[b],
            ).wait()

        gvecs = [(_iota16() + 16 * k) // 8 for k in range(4)]
        civecs = [(_iota16() + 16 * k) % 8 for k in range(4)]

        def transpose(b):
            def blk(i8, si_vec):
                base = i8 * 8
                vecs = [si_vec + j for j in range(8)]
                for j in range(8):
                    for k in range(4):
                        v = rows_v[b, base + j, pl.ds(16 * k, 16)]
                        plsc.store_scatter(
                            rowsT_v.at[b], [gvecs[k], civecs[k], vecs[j]], v
                        )
                return si_vec + 8

            lax.fori_loop(0, 16, blk, jnp.zeros((16,), jnp.int32))

        pltpu.sync_copy(xs3_hbm.at[:, wid], idx_v)

        for b in range(NB):
            start_gather(b, b)

        def step(q, b, first, last):
            r = q * NB + b
            wait_gather(b)
            if not first:
                wait_out(b)
            transpose(b)
            start_out(b, r)
            if not last:
                start_gather(b, r + NB)

        for b in range(NB):
            step(0, b, True, False)

        def round_body(q, _):
            for b in range(NB):
                step(q, b, False, False)
            return _

        lax.fori_loop(1, GU // NB - 1, round_body, None)

        for b in range(NB):
            step(GU // NB - 1, b, False, True)
        for b in range(NB):
            wait_out(b)

    return body(xs3, tl)


def kernel(x, table):
    xs3 = x.T.astype(jnp.int32).reshape(GU, NW, 128)
    out5 = _gather_t(xs3, table)
    return out5.transpose(2, 4, 0, 1, 3).reshape(S, T, D)


# speedup vs baseline: 1.0000x; 1.0000x over previous
import functools

import jax
import jax.numpy as jnp
from jax import lax
from jax.experimental import pallas as pl
from jax.experimental.pallas import tpu as pltpu
from jax.experimental.pallas import tpu_sc as plsc

NC = 2
NS = 16
NW = NC * NS

D = 64
T = 50
S = 16384
NB = 5
GU = (T * (S // 128)) // NW


def _iota16():
    return lax.iota(jnp.int32, 16)


@jax.jit
def _gather_t(xs3, tl):
    mesh = plsc.VectorSubcoreMesh(core_axis_name="c", subcore_axis_name="s")

    @functools.partial(
        pl.kernel,
        mesh=mesh,
        out_type=jax.ShapeDtypeStruct((T, 8, S // 128, 8, 128), jnp.float32),
        scratch_types=[
            pltpu.VMEM((GU, 128), jnp.int32),
            pltpu.VMEM((NB, 128, 64), jnp.float32),
            pltpu.VMEM((NB, 8, 8, 129), jnp.float32),
            pltpu.SemaphoreType.DMA((NB,)),
            pltpu.SemaphoreType.DMA((NB,)),
        ],
        compiler_params=pltpu.CompilerParams(
            use_tc_tiling_on_sc=False, needs_layout_passes=False
        ),
    )
    def body(xs3_hbm, tl_hbm, out_hbm, idx_v, rows_v, rowsT_v, gsems, osems):
        wid = lax.axis_index("s") * NC + lax.axis_index("c")

        def unit(r):
            u = r * NW + wid
            return u // 128, u % 128

        def start_gather(b, r):
            pltpu.async_copy(
                tl_hbm.at[idx_v.at[r]], rows_v.at[b], gsems.at[b]
            )

        def wait_gather(b):
            pltpu.make_async_copy(
                tl_hbm.at[pl.ds(0, 128)], rows_v.at[b], gsems.at[b]
            ).wait()

        def start_out(b, r):
            t, w = unit(r)
            pltpu.async_copy(
                rowsT_v.at[b, :, :, pl.ds(0, 128)],
                out_hbm.at[t, :, w],
                osems.at[b],
            )

        def wait_out(b):
            pltpu.make_async_copy(
                rowsT_v.at[b, :, :, pl.ds(0, 128)],
                out_hbm.at[0, :, 0],
                osems.at[b],
            ).wait()

        gvecs = [(_iota16() + 16 * k) // 8 for k in range(4)]
        civecs = [(_iota16() + 16 * k) % 8 for k in range(4)]

        def transpose(b):
            def blk(i8, si_vec):
                base = i8 * 8
                vecs = [si_vec + j for j in range(8)]
                for j in range(8):
                    for k in range(4):
                        v = rows_v[b, base + j, pl.ds(16 * k, 16)]
                        plsc.store_scatter(
                            rowsT_v.at[b], [gvecs[k], civecs[k], vecs[j]], v
                        )
                return si_vec + 8

            lax.fori_loop(0, 16, blk, jnp.zeros((16,), jnp.int32))

        pltpu.sync_copy(xs3_hbm.at[:, wid], idx_v)

        for b in range(NB):
            start_gather(b, b)

        def step(q, b, first, last):
            r = q * NB + b
            wait_gather(b)
            if not first:
                wait_out(b)
            transpose(b)
            start_out(b, r)
            if not last:
                start_gather(b, r + NB)

        for b in range(NB):
            step(0, b, True, False)

        def round_body(q, _):
            for b in range(NB):
                step(q, b, False, False)
            return _

        lax.fori_loop(1, GU // NB - 1, round_body, None)

        for b in range(NB):
            step(GU // NB - 1, b, False, True)
        for b in range(NB):
            wait_out(b)

    return body(xs3, tl)


def kernel(x, table):
    xs3 = x.T.astype(jnp.int32).reshape(GU, NW, 128)
    out5 = _gather_t(xs3, table)
    return out5.transpose(2, 4, 0, 1, 3).reshape(S, T, D)


# speedup vs baseline: 1.0684x; 1.0684x over previous
import functools

import jax
import jax.numpy as jnp
from jax import lax
from jax.experimental import pallas as pl
from jax.experimental.pallas import tpu as pltpu
from jax.experimental.pallas import tpu_sc as plsc

NC = 2
NS = 16
NW = NC * NS

D = 64
T = 50
S = 16384
NB = 4
GU = (T * (S // 128)) // NW


def _iota16():
    return lax.iota(jnp.int32, 16)


@jax.jit
def _gather_t(xs3, tl):
    mesh = plsc.VectorSubcoreMesh(core_axis_name="c", subcore_axis_name="s")

    @functools.partial(
        pl.kernel,
        mesh=mesh,
        out_type=jax.ShapeDtypeStruct((T, 8, S // 128, 8, 128), jnp.float32),
        scratch_types=[
            pltpu.VMEM((GU, 128), jnp.int32),
            pltpu.VMEM((NB, 128, 128), jnp.float32),
            pltpu.VMEM((NB, 8, 8, 129), jnp.float32),
            pltpu.SemaphoreType.DMA((NB,)),
            pltpu.SemaphoreType.DMA((NB,)),
        ],
        compiler_params=pltpu.CompilerParams(
            use_tc_tiling_on_sc=False, needs_layout_passes=False
        ),
    )
    def body(xs3_hbm, tl_hbm, out_hbm, idx_v, rows_v, rowsT_v, gsems, osems):
        wid = lax.axis_index("s") * NC + lax.axis_index("c")

        def unit(r):
            u = r * NW + wid
            return u // 128, u % 128

        def start_gather(b, r):
            pltpu.async_copy(
                tl_hbm.at[idx_v.at[r]], rows_v.at[b], gsems.at[b]
            )

        def wait_gather(b):
            pltpu.make_async_copy(
                tl_hbm.at[pl.ds(0, 128)], rows_v.at[b], gsems.at[b]
            ).wait()

        def start_out(b, r):
            t, w = unit(r)
            pltpu.async_copy(
                rowsT_v.at[b, :, :, pl.ds(0, 128)],
                out_hbm.at[t, :, w],
                osems.at[b],
            )

        def wait_out(b):
            pltpu.make_async_copy(
                rowsT_v.at[b, :, :, pl.ds(0, 128)],
                out_hbm.at[0, :, 0],
                osems.at[b],
            ).wait()

        gvecs = [(_iota16() + 16 * k) // 8 for k in range(4)]
        civecs = [(_iota16() + 16 * k) % 8 for k in range(4)]

        def transpose(b):
            def blk(i8, si_vec):
                base = i8 * 8
                vecs = [si_vec + j for j in range(8)]
                for j in range(8):
                    for k in range(4):
                        v = rows_v[b, base + j, pl.ds(16 * k, 16)]
                        plsc.store_scatter(
                            rowsT_v.at[b], [gvecs[k], civecs[k], vecs[j]], v
                        )
                return si_vec + 8

            lax.fori_loop(0, 16, blk, jnp.zeros((16,), jnp.int32))

        pltpu.sync_copy(xs3_hbm.at[:, wid], idx_v)

        for b in range(NB):
            start_gather(b, b)

        def step(q, b, first, last):
            r = q * NB + b
            wait_gather(b)
            if not first:
                wait_out(b)
            transpose(b)
            start_out(b, r)
            if not last:
                start_gather(b, r + NB)

        for b in range(NB):
            step(0, b, True, False)

        def round_body(q, _):
            for b in range(NB):
                step(q, b, False, False)
            return _

        lax.fori_loop(1, GU // NB - 1, round_body, None)

        for b in range(NB):
            step(GU // NB - 1, b, False, True)
        for b in range(NB):
            wait_out(b)

    return body(xs3, tl)


def kernel(x, table):
    xs3 = x.T.astype(jnp.int32).reshape(GU, NW, 128)
    tp = jnp.pad(table, ((0, 0), (0, D)))
    out5 = _gather_t(xs3, tp)
    return out5.transpose(2, 4, 0, 1, 3).reshape(S, T, D)
